# baseline (speedup 1.0000x reference)
import functools

import jax
import jax.numpy as jnp
from jax import lax
from jax.experimental import pallas as pl
from jax.experimental.pallas import tpu as pltpu
from jax.experimental.pallas import tpu_sc as plsc

N_VOCAB = 1000000
N_EMBED = 64
B_TOK = 4096 * 200

_NC = 2
_NS = 16
_NW = _NC * _NS
_BPW = B_TOK // _NW
_CHUNK = 512
_NSTEP = _BPW // _CHUNK

_mesh = plsc.VectorSubcoreMesh(core_axis_name="c", subcore_axis_name="s")


@functools.partial(
    pl.kernel,
    mesh=_mesh,
    compiler_params=pltpu.CompilerParams(use_tc_tiling_on_sc=False),
    out_type=jax.ShapeDtypeStruct((B_TOK, N_EMBED), jnp.float32),
    scratch_types=[
        pltpu.VMEM((_CHUNK,), jnp.int32),
        pltpu.VMEM((_CHUNK, N_EMBED), jnp.float32),
        pltpu.SemaphoreType.DMA,
    ],
)
def _embed_gather(idx_hbm, table_hbm, out_hbm, idx_v, rows_v, sem):
    wid = lax.axis_index("s") * _NC + lax.axis_index("c")
    base = wid * _BPW

    def step(ci, _):
        off = base + ci * _CHUNK
        pltpu.sync_copy(idx_hbm.at[pl.ds(off, _CHUNK)], idx_v)
        pltpu.async_copy(table_hbm.at[idx_v], rows_v, sem).wait()
        pltpu.sync_copy(rows_v, out_hbm.at[pl.ds(off, _CHUNK)])
        return ()

    lax.fori_loop(0, _NSTEP, step, (), unroll=False)


def kernel(x, table):
    flat = x.reshape(B_TOK)
    out = _embed_gather(flat, table)
    emb = out.reshape(x.shape[0], x.shape[1], N_EMBED)
    return (emb, emb)


# speedup vs baseline: 1.0426x; 1.0426x over previous
import functools

import jax
import jax.numpy as jnp
from jax import lax
from jax.experimental import pallas as pl
from jax.experimental.pallas import tpu as pltpu
from jax.experimental.pallas import tpu_sc as plsc

N_VOCAB = 1000000
N_EMBED = 64
B_TOK = 4096 * 200

_NC = 2
_NS = 16
_NW = _NC * _NS
_BPW = B_TOK // _NW
_CHUNK = 512
_NSTEP = _BPW // _CHUNK
_NBUF = 2

_mesh = plsc.VectorSubcoreMesh(core_axis_name="c", subcore_axis_name="s")


@functools.partial(
    pl.kernel,
    mesh=_mesh,
    compiler_params=pltpu.CompilerParams(use_tc_tiling_on_sc=False),
    out_type=jax.ShapeDtypeStruct((B_TOK, N_EMBED), jnp.float32),
    scratch_types=[
        pltpu.VMEM((_BPW,), jnp.int32),
        pltpu.VMEM((_CHUNK, N_EMBED), jnp.float32),
        pltpu.VMEM((_CHUNK, N_EMBED), jnp.float32),
        pltpu.SemaphoreType.DMA,
        pltpu.SemaphoreType.DMA,
        pltpu.SemaphoreType.DMA,
        pltpu.SemaphoreType.DMA,
    ],
)
def _embed_gather(idx_hbm, table_hbm, out_hbm, idx_v, r0, r1, g0, g1, w0, w1):
    rows = (r0, r1)
    gsem = (g0, g1)
    wsem = (w0, w1)
    wid = lax.axis_index("s") * _NC + lax.axis_index("c")
    base = wid * _BPW

    pltpu.sync_copy(idx_hbm.at[pl.ds(base, _BPW)], idx_v)

    def g_start(b, ci):
        pltpu.async_copy(
            table_hbm.at[idx_v.at[pl.ds(ci * _CHUNK, _CHUNK)]], rows[b], gsem[b])

    def g_wait(b):
        pltpu.make_async_copy(
            table_hbm.at[idx_v.at[pl.ds(0, _CHUNK)]], rows[b], gsem[b]).wait()

    def w_start(b, ci):
        pltpu.async_copy(
            rows[b], out_hbm.at[pl.ds(base + ci * _CHUNK, _CHUNK)], wsem[b])

    def w_wait(b):
        pltpu.make_async_copy(
            rows[b], out_hbm.at[pl.ds(base, _CHUNK)], wsem[b]).wait()

    for b in range(_NBUF):
        g_start(b, b)

    def loop_body(i, _):
        for b in range(_NBUF):
            ci = i * _NBUF + b
            g_wait(b)
            w_start(b, ci)
            w_wait(b)
            g_start(b, ci + _NBUF)
        return ()

    lax.fori_loop(0, _NSTEP // _NBUF - 1, loop_body, (), unroll=False)

    for b in range(_NBUF):
        ci = _NSTEP - _NBUF + b
        g_wait(b)
        w_start(b, ci)
        w_wait(b)


def kernel(x, table):
    flat = x.reshape(B_TOK)
    out = _embed_gather(flat, table)
    emb = out.reshape(x.shape[0], x.shape[1], N_EMBED)
    return (emb, emb)
